# baseline (speedup 1.0000x reference)
import functools

import jax
import jax.numpy as jnp
from jax import lax
from jax.experimental import pallas as pl
from jax.experimental.pallas import tpu as pltpu
from jax.experimental.pallas import tpu_sc as plsc

N = 10000
E = 320000
D = 128

NC, NS = 2, 16
NW = NC * NS
ROWS_PER_TILE = 640
NPAD = NS * ROWS_PER_TILE
CHUNK = 128
TOTAL_CHUNKS = 2560
EPAD = TOTAL_CHUNKS * CHUNK

CPT0, CPT1 = 144, 16
STAGES0 = (48, 48, 48)
STAGES1 = (16,)
IDXBUF = max(max(STAGES0), max(STAGES1))

DEG_CPT = TOTAL_CHUNKS // NW

_mesh = plsc.VectorSubcoreMesh(
    core_axis_name="c", subcore_axis_name="s", num_cores=NC, num_subcores=NS
)


@functools.partial(
    pl.kernel,
    out_type=jax.ShapeDtypeStruct((NC, NPAD), jnp.float32),
    mesh=_mesh,
    scratch_types=[
        pltpu.VMEM((DEG_CPT, CHUNK), jnp.int32),
        pltpu.VMEM((CHUNK,), jnp.float32),
        pltpu.VMEM_SHARED((NPAD,), jnp.float32),
    ],
)
def _deg_kernel(dst_hbm, ones_hbm, z_hbm, out_hbm, dst_v, ones_v, acc):
    c = lax.axis_index("c")
    s = lax.axis_index("s")
    wid = c * NS + s
    rows = pl.ds(s * ROWS_PER_TILE, ROWS_PER_TILE)
    pltpu.sync_copy(dst_hbm.at[pl.ds(wid * DEG_CPT, DEG_CPT)], dst_v)
    pltpu.sync_copy(ones_hbm, ones_v)
    pltpu.sync_copy(z_hbm.at[rows], acc.at[rows])
    plsc.subcore_barrier()

    def body(j, carry):
        pltpu.sync_copy(ones_v, acc.at[dst_v.at[j]], add=True)
        return carry

    lax.fori_loop(0, DEG_CPT, body, 0)
    plsc.subcore_barrier()
    pltpu.sync_copy(acc.at[rows], out_hbm.at[c].at[rows])


@functools.partial(
    pl.kernel,
    out_type=jax.ShapeDtypeStruct((NC, NPAD, D), jnp.float32),
    mesh=_mesh,
    scratch_types=[
        pltpu.VMEM((IDXBUF, CHUNK), jnp.int32),
        pltpu.VMEM((IDXBUF, CHUNK), jnp.int32),
        pltpu.VMEM((2, CHUNK, D), jnp.float32),
        pltpu.VMEM_SHARED((NPAD, D), jnp.float32),
        pltpu.SemaphoreType.DMA,
    ],
)
def _scatter_kernel(h_hbm, src_hbm, dst_hbm, z_hbm, out_hbm,
                    src_v, dst_v, rows_v, acc, sem):
    c = lax.axis_index("c")
    s = lax.axis_index("s")
    rows = pl.ds(s * ROWS_PER_TILE, ROWS_PER_TILE)
    pltpu.sync_copy(z_hbm, rows_v.at[0])
    for k in range(ROWS_PER_TILE // CHUNK):
        pltpu.sync_copy(rows_v.at[0],
                        acc.at[pl.ds(s * ROWS_PER_TILE + k * CHUNK, CHUNK)])
    plsc.subcore_barrier()

    def run_side(side_base, cpt, stages):
        tile_base = side_base + s * cpt
        off = 0
        for stage in stages:
            chunks = pl.ds(tile_base + off, stage)
            off += stage
            idx = pl.ds(0, stage)
            pltpu.sync_copy(src_hbm.at[chunks], src_v.at[idx])
            pltpu.sync_copy(dst_hbm.at[chunks], dst_v.at[idx])
            pltpu.async_copy(h_hbm.at[src_v.at[0]], rows_v.at[0], sem)

            def body(j, carry):
                b = lax.rem(j, 2)
                pltpu.make_async_copy(h_hbm.at[src_v.at[j]], rows_v.at[b], sem).wait()

                @pl.when(j < stage - 1)
                def _():
                    pltpu.async_copy(h_hbm.at[src_v.at[j + 1]], rows_v.at[1 - b], sem)

                pltpu.sync_copy(rows_v.at[b], acc.at[dst_v.at[j]], add=True)
                return carry

            lax.fori_loop(0, stage, body, 0)

    @pl.when(c == 1)
    def _():
        run_side(0, CPT0, STAGES0)

    @pl.when(c == 0)
    def _():
        run_side(NS * CPT0, CPT1, STAGES1)

    plsc.subcore_barrier()
    pltpu.sync_copy(acc.at[rows], out_hbm.at[c].at[rows])


_R = 400


def _dis_block(deg_ref):
    v = deg_ref[...]
    deg = v[0] + v[1] + 1.0
    return lax.rsqrt(deg)


def _prep_body(deg_ref, x_ref, w_ref, out_ref):
    dis = _dis_block(deg_ref)
    h = jnp.dot(x_ref[...], w_ref[...], preferred_element_type=jnp.float32)
    out_ref[...] = h * dis


def _mid_body(deg_ref, s_ref, h_ref, b_ref, w_ref, out_ref):
    dis = _dis_block(deg_ref)
    sv = s_ref[...]
    z = dis * (sv[0] + sv[1] + h_ref[...]) + b_ref[...]
    z = jnp.maximum(z, 0.0)
    out_ref[...] = jnp.dot(z, w_ref[...], preferred_element_type=jnp.float32) * dis


def _final_body(deg_ref, s_ref, h_ref, b_ref, out_ref):
    dis = _dis_block(deg_ref)
    sv = s_ref[...]
    z = dis * (sv[0] + sv[1] + h_ref[...]) + b_ref[...]
    out_ref[...] = jnp.maximum(z, 0.0)


_deg_spec = pl.BlockSpec((NC, _R, 1), lambda i: (0, i, 0))
_row_spec = pl.BlockSpec((_R, D), lambda i: (i, 0))
_part_spec = pl.BlockSpec((NC, _R, D), lambda i: (0, i, 0))
_w_spec = pl.BlockSpec((D, D), lambda i: (0, 0))
_b_spec = pl.BlockSpec((1, D), lambda i: (0, 0))
_out_shape = jax.ShapeDtypeStruct((N, D), jnp.float32)


def _tc_prep(degp, x, w1):
    return pl.pallas_call(
        _prep_body,
        grid=(N // _R,),
        in_specs=[_deg_spec, _row_spec, _w_spec],
        out_specs=_row_spec,
        out_shape=_out_shape,
    )(degp, x, w1)


def _tc_mid(degp, s1, h1s, b1, w2):
    return pl.pallas_call(
        _mid_body,
        grid=(N // _R,),
        in_specs=[_deg_spec, _part_spec, _row_spec, _b_spec, _w_spec],
        out_specs=_row_spec,
        out_shape=_out_shape,
    )(degp, s1, h1s, b1, w2)


def _tc_final(degp, s2, h2s, b2):
    return pl.pallas_call(
        _final_body,
        grid=(N // _R,),
        in_specs=[_deg_spec, _part_spec, _row_spec, _b_spec],
        out_specs=_row_spec,
        out_shape=_out_shape,
    )(degp, s2, h2s, b2)


@jax.jit
def _gcn(x, edge_index, w1, b1, w2, b2):
    src = edge_index[0].astype(jnp.int32)
    dst = edge_index[1].astype(jnp.int32)
    pad = EPAD - E
    src_p = jnp.concatenate([src, jnp.zeros((pad,), jnp.int32)])
    dst_p = jnp.concatenate([dst, jnp.full((pad,), N, jnp.int32)])
    src_p = src_p.reshape(TOTAL_CHUNKS, CHUNK)
    dst_p = dst_p.reshape(TOTAL_CHUNKS, CHUNK)
    ones1 = jnp.ones((CHUNK,), jnp.float32)
    z1 = jnp.zeros((NPAD,), jnp.float32)
    z128 = jnp.zeros((CHUNK, D), jnp.float32)
    b1r = b1.reshape(1, D)
    b2r = b2.reshape(1, D)

    degp = _deg_kernel(dst_p, ones1, z1).reshape(NC, NPAD, 1)
    h1s = _tc_prep(degp, x, w1)
    s1 = _scatter_kernel(h1s, src_p, dst_p, z128)
    h2s = _tc_mid(degp, s1, h1s, b1r, w2)
    s2 = _scatter_kernel(h2s, src_p, dst_p, z128)
    return _tc_final(degp, s2, h2s, b2r)


def kernel(x, edge_index, W1, b1, W2, b2):
    return _gcn(x, edge_index, W1, b1, W2, b2)


# speedup vs baseline: 1.0028x; 1.0028x over previous
import functools

import jax
import jax.numpy as jnp
from jax import lax
from jax.experimental import pallas as pl
from jax.experimental.pallas import tpu as pltpu
from jax.experimental.pallas import tpu_sc as plsc

N = 10000
E = 320000
D = 128

NC, NS = 2, 16
NW = NC * NS
ROWS_PER_TILE = 640
NPAD = NS * ROWS_PER_TILE
CHUNK = 128
TOTAL_CHUNKS = 2560
EPAD = TOTAL_CHUNKS * CHUNK

CPT0, CPT1 = 144, 16
STAGES0 = (48, 48, 48)
STAGES1 = (16,)
IDXBUF = max(max(STAGES0), max(STAGES1))

DEG_CPT = TOTAL_CHUNKS // NW

_mesh = plsc.VectorSubcoreMesh(
    core_axis_name="c", subcore_axis_name="s", num_cores=NC, num_subcores=NS
)


@functools.partial(
    pl.kernel,
    out_type=jax.ShapeDtypeStruct((NC, NPAD), jnp.float32),
    mesh=_mesh,
    scratch_types=[
        pltpu.VMEM((DEG_CPT, CHUNK), jnp.int32),
        pltpu.VMEM((CHUNK,), jnp.float32),
        pltpu.VMEM_SHARED((NPAD,), jnp.float32),
    ],
)
def _deg_kernel(dst_hbm, ones_hbm, z_hbm, out_hbm, dst_v, ones_v, acc):
    c = lax.axis_index("c")
    s = lax.axis_index("s")
    wid = c * NS + s
    rows = pl.ds(s * ROWS_PER_TILE, ROWS_PER_TILE)
    pltpu.sync_copy(dst_hbm.at[pl.ds(wid * DEG_CPT, DEG_CPT)], dst_v)
    pltpu.sync_copy(ones_hbm, ones_v)
    pltpu.sync_copy(z_hbm.at[rows], acc.at[rows])
    plsc.subcore_barrier()

    def body(j, carry):
        pltpu.sync_copy(ones_v, acc.at[dst_v.at[j]], add=True)
        return carry

    lax.fori_loop(0, DEG_CPT, body, 0)
    plsc.subcore_barrier()
    pltpu.sync_copy(acc.at[rows], out_hbm.at[c].at[rows])


@functools.partial(
    pl.kernel,
    out_type=jax.ShapeDtypeStruct((NC, NPAD, D), jnp.float32),
    mesh=_mesh,
    scratch_types=[
        pltpu.VMEM((IDXBUF, CHUNK), jnp.int32),
        pltpu.VMEM((IDXBUF, CHUNK), jnp.int32),
        pltpu.VMEM((2, CHUNK, D), jnp.float32),
        pltpu.VMEM_SHARED((NPAD, D), jnp.float32),
        pltpu.SemaphoreType.DMA,
    ],
)
def _scatter_kernel(h_hbm, src_hbm, dst_hbm, z_hbm, out_hbm,
                    src_v, dst_v, rows_v, acc, sem):
    c = lax.axis_index("c")
    s = lax.axis_index("s")
    rows = pl.ds(s * ROWS_PER_TILE, ROWS_PER_TILE)
    pltpu.sync_copy(z_hbm, rows_v.at[0])
    for k in range(ROWS_PER_TILE // CHUNK):
        pltpu.sync_copy(rows_v.at[0],
                        acc.at[pl.ds(s * ROWS_PER_TILE + k * CHUNK, CHUNK)])
    plsc.subcore_barrier()

    def run_side(side_base, cpt, stages):
        tile_base = side_base + s * cpt
        off = 0
        for stage in stages:
            chunks = pl.ds(tile_base + off, stage)
            off += stage
            idx = pl.ds(0, stage)
            pltpu.sync_copy(src_hbm.at[chunks], src_v.at[idx])
            pltpu.sync_copy(dst_hbm.at[chunks], dst_v.at[idx])
            pltpu.async_copy(h_hbm.at[src_v.at[0]], rows_v.at[0], sem)

            def body(j, carry):
                b = lax.rem(j, 2)
                pltpu.make_async_copy(h_hbm.at[src_v.at[j]], rows_v.at[b], sem).wait()

                @pl.when(j < stage - 1)
                def _():
                    pltpu.async_copy(h_hbm.at[src_v.at[j + 1]], rows_v.at[1 - b], sem)

                pltpu.sync_copy(rows_v.at[b], acc.at[dst_v.at[j]], add=True)
                return carry

            lax.fori_loop(0, stage, body, 0)

    @pl.when(c == 0)
    def _():
        run_side(0, CPT0, STAGES0)

    @pl.when(c == 1)
    def _():
        run_side(NS * CPT0, CPT1, STAGES1)

    plsc.subcore_barrier()
    pltpu.sync_copy(acc.at[rows], out_hbm.at[c].at[rows])


_R = 400


def _dis_block(deg_ref):
    v = deg_ref[...]
    deg = v[0] + v[1] + 1.0
    return lax.rsqrt(deg)


def _prep_body(deg_ref, x_ref, w_ref, out_ref):
    dis = _dis_block(deg_ref)
    h = jnp.dot(x_ref[...], w_ref[...], preferred_element_type=jnp.float32)
    out_ref[...] = h * dis


def _mid_body(deg_ref, s_ref, h_ref, b_ref, w_ref, out_ref):
    dis = _dis_block(deg_ref)
    sv = s_ref[...]
    z = dis * (sv[0] + sv[1] + h_ref[...]) + b_ref[...]
    z = jnp.maximum(z, 0.0)
    out_ref[...] = jnp.dot(z, w_ref[...], preferred_element_type=jnp.float32) * dis


def _final_body(deg_ref, s_ref, h_ref, b_ref, out_ref):
    dis = _dis_block(deg_ref)
    sv = s_ref[...]
    z = dis * (sv[0] + sv[1] + h_ref[...]) + b_ref[...]
    out_ref[...] = jnp.maximum(z, 0.0)


_deg_spec = pl.BlockSpec((NC, _R, 1), lambda i: (0, i, 0))
_row_spec = pl.BlockSpec((_R, D), lambda i: (i, 0))
_part_spec = pl.BlockSpec((NC, _R, D), lambda i: (0, i, 0))
_w_spec = pl.BlockSpec((D, D), lambda i: (0, 0))
_b_spec = pl.BlockSpec((1, D), lambda i: (0, 0))
_out_shape = jax.ShapeDtypeStruct((N, D), jnp.float32)


def _tc_prep(degp, x, w1):
    return pl.pallas_call(
        _prep_body,
        grid=(N // _R,),
        in_specs=[_deg_spec, _row_spec, _w_spec],
        out_specs=_row_spec,
        out_shape=_out_shape,
    )(degp, x, w1)


def _tc_mid(degp, s1, h1s, b1, w2):
    return pl.pallas_call(
        _mid_body,
        grid=(N // _R,),
        in_specs=[_deg_spec, _part_spec, _row_spec, _b_spec, _w_spec],
        out_specs=_row_spec,
        out_shape=_out_shape,
    )(degp, s1, h1s, b1, w2)


def _tc_final(degp, s2, h2s, b2):
    return pl.pallas_call(
        _final_body,
        grid=(N // _R,),
        in_specs=[_deg_spec, _part_spec, _row_spec, _b_spec],
        out_specs=_row_spec,
        out_shape=_out_shape,
    )(degp, s2, h2s, b2)


@jax.jit
def _gcn(x, edge_index, w1, b1, w2, b2):
    src = edge_index[0].astype(jnp.int32)
    dst = edge_index[1].astype(jnp.int32)
    pad = EPAD - E
    src_p = jnp.concatenate([src, jnp.zeros((pad,), jnp.int32)])
    dst_p = jnp.concatenate([dst, jnp.full((pad,), N, jnp.int32)])
    src_p = src_p.reshape(TOTAL_CHUNKS, CHUNK)
    dst_p = dst_p.reshape(TOTAL_CHUNKS, CHUNK)
    ones1 = jnp.ones((CHUNK,), jnp.float32)
    z1 = jnp.zeros((NPAD,), jnp.float32)
    z128 = jnp.zeros((CHUNK, D), jnp.float32)
    b1r = b1.reshape(1, D)
    b2r = b2.reshape(1, D)

    degp = _deg_kernel(dst_p, ones1, z1).reshape(NC, NPAD, 1)
    h1s = _tc_prep(degp, x, w1)
    s1 = _scatter_kernel(h1s, src_p, dst_p, z128)
    h2s = _tc_mid(degp, s1, h1s, b1r, w2)
    s2 = _scatter_kernel(h2s, src_p, dst_p, z128)
    return _tc_final(degp, s2, h2s, b2r)


def kernel(x, edge_index, W1, b1, W2, b2):
    return _gcn(x, edge_index, W1, b1, W2, b2)
